# Initial kernel scaffold; baseline (speedup 1.0000x reference)
import jax
import jax.numpy as jnp
from jax.experimental import pallas as pl


def kernel(x, edge_index, W1, b1, W2, b2, W3, b3):
    raise NotImplementedError("write your pallas kernel here")


# speedup vs baseline: 1.0685x; 1.0685x over previous
import functools
import jax
import jax.numpy as jnp
from jax.experimental import pallas as pl
from jax.experimental.pallas import tpu as pltpu

N = 10000
E = 320000
F = 128
H = 128
C = 47
CP = 64

BLK = 1000


def _dense_body(x_ref, a_ref, wa_ref, wb_ref, b_ref, o_ref):
    z = jnp.dot(x_ref[...], wa_ref[...], preferred_element_type=jnp.float32)
    z = z + jnp.dot(a_ref[...], wb_ref[...], preferred_element_type=jnp.float32)
    z = z + b_ref[...]
    o_ref[...] = jnp.maximum(z, 0.0)


def _dense_relu(x, aggr, wa, wb, b):
    grid = (N // BLK,)
    return pl.pallas_call(
        _dense_body,
        grid=grid,
        in_specs=[
            pl.BlockSpec((BLK, F), lambda i: (i, 0)),
            pl.BlockSpec((BLK, F), lambda i: (i, 0)),
            pl.BlockSpec((F, H), lambda i: (0, 0)),
            pl.BlockSpec((F, H), lambda i: (0, 0)),
            pl.BlockSpec((1, H), lambda i: (0, 0)),
        ],
        out_specs=pl.BlockSpec((BLK, H), lambda i: (i, 0)),
        out_shape=jax.ShapeDtypeStruct((N, H), jnp.float32),
    )(x, aggr, wa, wb, b.reshape(1, H))


def _dense_relu_pre_body(x_ref, a_ref, wa_ref, wb_ref, b_ref, wn_ref, o_ref, t_ref):
    z = jnp.dot(x_ref[...], wa_ref[...], preferred_element_type=jnp.float32)
    z = z + jnp.dot(a_ref[...], wb_ref[...], preferred_element_type=jnp.float32)
    z = z + b_ref[...]
    h = jnp.maximum(z, 0.0)
    o_ref[...] = h
    t_ref[...] = jnp.dot(h, wn_ref[...], preferred_element_type=jnp.float32)


def _dense_relu_premul(x, aggr, wa, wb, b, w_next):
    grid = (N // BLK,)
    return pl.pallas_call(
        _dense_relu_pre_body,
        grid=grid,
        in_specs=[
            pl.BlockSpec((BLK, H), lambda i: (i, 0)),
            pl.BlockSpec((BLK, H), lambda i: (i, 0)),
            pl.BlockSpec((H, H), lambda i: (0, 0)),
            pl.BlockSpec((H, H), lambda i: (0, 0)),
            pl.BlockSpec((1, H), lambda i: (0, 0)),
            pl.BlockSpec((H, CP), lambda i: (0, 0)),
        ],
        out_specs=[
            pl.BlockSpec((BLK, H), lambda i: (i, 0)),
            pl.BlockSpec((BLK, CP), lambda i: (i, 0)),
        ],
        out_shape=[
            jax.ShapeDtypeStruct((N, H), jnp.float32),
            jax.ShapeDtypeStruct((N, CP), jnp.float32),
        ],
    )(x, aggr, wa, wb, b.reshape(1, H), w_next)


def _final_body(x_ref, q_ref, wa_ref, b_ref, o_ref):
    z = jnp.dot(x_ref[...], wa_ref[...], preferred_element_type=jnp.float32)
    z = z + q_ref[...] + b_ref[...]
    col = jax.lax.broadcasted_iota(jnp.int32, (BLK, CP), 1)
    z = jnp.where(col < C, z, -jnp.inf)
    m = jnp.max(z, axis=-1, keepdims=True)
    lse = jnp.log(jnp.sum(jnp.exp(z - m), axis=-1, keepdims=True))
    o_ref[...] = z - m - lse


def _final(h, q, wa, b):
    grid = (N // BLK,)
    return pl.pallas_call(
        _final_body,
        grid=grid,
        in_specs=[
            pl.BlockSpec((BLK, H), lambda i: (i, 0)),
            pl.BlockSpec((BLK, CP), lambda i: (i, 0)),
            pl.BlockSpec((H, CP), lambda i: (0, 0)),
            pl.BlockSpec((1, CP), lambda i: (0, 0)),
        ],
        out_specs=pl.BlockSpec((BLK, CP), lambda i: (i, 0)),
        out_shape=jax.ShapeDtypeStruct((N, CP), jnp.float32),
    )(h, q, wa, b)


def kernel(x, edge_index, W1, b1, W2, b2, W3, b3):
    src = edge_index[0]
    dst = edge_index[1]

    W1a, W1b = W1[:F], W1[F:]
    W2a, W2b = W2[:H], W2[H:]
    W3a = jnp.pad(W3[:H], ((0, 0), (0, CP - C)))
    W3b = jnp.pad(W3[H:], ((0, 0), (0, CP - C)))
    b3p = jnp.pad(b3, (0, CP - C)).reshape(1, CP)

    aggr1 = jax.ops.segment_sum(x[src], dst, num_segments=N)
    h1 = _dense_relu(x, aggr1, W1a, W1b, b1)

    aggr2 = jax.ops.segment_sum(h1[src], dst, num_segments=N)
    h2, t3 = _dense_relu_premul(h1, aggr2, W2a, W2b, b2, W3b)

    q3 = jax.ops.segment_sum(t3[src], dst, num_segments=N)
    out = _final(h2, q3, W3a, b3p)
    return out[:, :C]


# speedup vs baseline: 6.3101x; 5.9056x over previous
import functools
import jax
import jax.numpy as jnp
from jax import lax
from jax.experimental import pallas as pl
from jax.experimental.pallas import tpu as pltpu
from jax.experimental.pallas import tpu_sc as plsc

N = 10000
E = 320000
F = 128
H = 128
C = 47
CP = 64

BLK = 1000

_info = plsc.get_sparse_core_info()
NC = _info.num_cores
NS = _info.num_subcores
NW = NC * NS
EP = E // NW
CHUNK = 80
NCH = EP // CHUNK
NP = 10240
RP = NP // NS


def _make_seg_sum(D):
    mesh = plsc.VectorSubcoreMesh(core_axis_name="c", subcore_axis_name="s")

    @functools.partial(
        pl.kernel,
        mesh=mesh,
        out_type=jax.ShapeDtypeStruct((NC * NP, D), jnp.float32),
        scratch_types=[
            pltpu.VMEM((NCH, CHUNK), jnp.int32),
            pltpu.VMEM((NCH, CHUNK), jnp.int32),
            pltpu.VMEM((CHUNK, D), jnp.float32),
            pltpu.VMEM_SHARED((NP, D), jnp.float32),
            pltpu.SemaphoreType.DMA,
        ],
    )
    def seg_sum(vals_hbm, src_hbm, dst_hbm, zeros_hbm, out_hbm,
                src_v, dst_v, rows_v, acc, sem):
        cid = lax.axis_index("c")
        sid = lax.axis_index("s")
        wid = sid * NC + cid
        pltpu.sync_copy(zeros_hbm.at[pl.ds(sid * RP, RP)],
                        acc.at[pl.ds(sid * RP, RP)])
        pltpu.sync_copy(src_hbm.at[wid], src_v)
        pltpu.sync_copy(dst_hbm.at[wid], dst_v)
        plsc.subcore_barrier()

        def body(i, carry):
            pltpu.async_copy(vals_hbm.at[src_v.at[i]], rows_v, sem).wait()
            pltpu.sync_copy(rows_v, acc.at[dst_v.at[i]], add=True)
            return carry

        lax.fori_loop(0, NCH, body, 0)
        plsc.subcore_barrier()
        pltpu.sync_copy(acc.at[pl.ds(sid * RP, RP)],
                        out_hbm.at[pl.ds(cid * NP + sid * RP, RP)])

    return seg_sum


_seg_sum_full = _make_seg_sum(H)


def _dense_body(x_ref, p0_ref, p1_ref, wa_ref, wb_ref, b_ref, o_ref):
    z = jnp.dot(x_ref[...], wa_ref[...], preferred_element_type=jnp.float32)
    z = z + jnp.dot(p0_ref[...] + p1_ref[...], wb_ref[...],
                    preferred_element_type=jnp.float32)
    z = z + b_ref[...]
    o_ref[...] = jnp.maximum(z, 0.0)


def _dense_relu(x, p0, p1, wa, wb, b):
    grid = (N // BLK,)
    return pl.pallas_call(
        _dense_body,
        grid=grid,
        in_specs=[
            pl.BlockSpec((BLK, F), lambda i: (i, 0)),
            pl.BlockSpec((BLK, F), lambda i: (i, 0)),
            pl.BlockSpec((BLK, F), lambda i: (i, 0)),
            pl.BlockSpec((F, H), lambda i: (0, 0)),
            pl.BlockSpec((F, H), lambda i: (0, 0)),
            pl.BlockSpec((1, H), lambda i: (0, 0)),
        ],
        out_specs=pl.BlockSpec((BLK, H), lambda i: (i, 0)),
        out_shape=jax.ShapeDtypeStruct((N, H), jnp.float32),
    )(x, p0, p1, wa, wb, b.reshape(1, H))


def _final_body(x_ref, q0_ref, q1_ref, wa_ref, wb_ref, b_ref, o_ref):
    z = jnp.dot(x_ref[...], wa_ref[...], preferred_element_type=jnp.float32)
    z = z + jnp.dot(q0_ref[...] + q1_ref[...], wb_ref[...],
                    preferred_element_type=jnp.float32)
    z = z + b_ref[...]
    col = jax.lax.broadcasted_iota(jnp.int32, (BLK, CP), 1)
    z = jnp.where(col < C, z, -jnp.inf)
    m = jnp.max(z, axis=-1, keepdims=True)
    lse = jnp.log(jnp.sum(jnp.exp(z - m), axis=-1, keepdims=True))
    o_ref[...] = z - m - lse


def _final(h, q0, q1, wa, wb, b):
    grid = (N // BLK,)
    return pl.pallas_call(
        _final_body,
        grid=grid,
        in_specs=[
            pl.BlockSpec((BLK, H), lambda i: (i, 0)),
            pl.BlockSpec((BLK, H), lambda i: (i, 0)),
            pl.BlockSpec((BLK, H), lambda i: (i, 0)),
            pl.BlockSpec((H, CP), lambda i: (0, 0)),
            pl.BlockSpec((H, CP), lambda i: (0, 0)),
            pl.BlockSpec((1, CP), lambda i: (0, 0)),
        ],
        out_specs=pl.BlockSpec((BLK, CP), lambda i: (i, 0)),
        out_shape=jax.ShapeDtypeStruct((N, CP), jnp.float32),
    )(h, q0, q1, wa, wb, b)


def kernel(x, edge_index, W1, b1, W2, b2, W3, b3):
    src = edge_index[0].reshape(NW, NCH, CHUNK)
    dst = edge_index[1].reshape(NW, NCH, CHUNK)

    W1a, W1b = W1[:F], W1[F:]
    W2a, W2b = W2[:H], W2[H:]
    W3a = jnp.pad(W3[:H], ((0, 0), (0, CP - C)))
    W3b = jnp.pad(W3[H:], ((0, 0), (0, CP - C)))
    b3p = jnp.pad(b3, (0, CP - C)).reshape(1, CP)

    zeros_h = jnp.zeros((NP, H), jnp.float32)

    p1 = _seg_sum_full(x, src, dst, zeros_h)
    h1 = _dense_relu(x, p1[:N], p1[NP:NP + N], W1a, W1b, b1)

    p2 = _seg_sum_full(h1, src, dst, zeros_h)
    h2 = _dense_relu(h1, p2[:N], p2[NP:NP + N], W2a, W2b, b2)

    p3 = _seg_sum_full(h2, src, dst, zeros_h)
    out = _final(h2, p3[:N], p3[NP:NP + N], W3a, W3b, b3p)
    return out[:, :C]


# speedup vs baseline: 9.7680x; 1.5480x over previous
import functools
import jax
import jax.numpy as jnp
from jax import lax
from jax.experimental import pallas as pl
from jax.experimental.pallas import tpu as pltpu
from jax.experimental.pallas import tpu_sc as plsc

N = 10000
E = 320000
F = 128
H = 128
C = 47
CP = 64

BLK = 1000

_info = plsc.get_sparse_core_info()
NC = _info.num_cores
NS = _info.num_subcores
NW = NC * NS
CHUNK = 128
NCH = 80
E2 = NW * NCH * CHUNK
NP = 10240
RP = NP // NS


def _make_seg_sum(D):
    mesh = plsc.VectorSubcoreMesh(core_axis_name="c", subcore_axis_name="s")

    @functools.partial(
        pl.kernel,
        mesh=mesh,
        out_type=jax.ShapeDtypeStruct((NC * NP, D), jnp.float32),
        scratch_types=[
            pltpu.VMEM((CHUNK,), jnp.int32),
            pltpu.VMEM((CHUNK,), jnp.int32),
            pltpu.VMEM((CHUNK,), jnp.int32),
            pltpu.VMEM((CHUNK,), jnp.int32),
            pltpu.VMEM((CHUNK, D), jnp.float32),
            pltpu.VMEM((CHUNK, D), jnp.float32),
            pltpu.VMEM_SHARED((NP, D), jnp.float32),
            pltpu.SemaphoreType.DMA,
            pltpu.SemaphoreType.DMA,
            pltpu.SemaphoreType.DMA,
            pltpu.SemaphoreType.DMA,
        ],
    )
    def seg_sum(vals_hbm, src_hbm, dst_hbm, zeros_hbm, out_hbm,
                src_a, dst_a, src_b, dst_b, rows_a, rows_b, acc,
                sem_a, sem_b, isem_a, isem_b):
        cid = lax.axis_index("c")
        sid = lax.axis_index("s")
        wid = sid * NC + cid
        base = wid * (NCH * CHUNK)
        pltpu.sync_copy(zeros_hbm.at[pl.ds(sid * RP, RP)],
                        acc.at[pl.ds(sid * RP, RP)])

        def idx_load(j, sbuf, dbuf, isem):
            off = base + j * CHUNK
            pltpu.async_copy(src_hbm.at[pl.ds(off, CHUNK)], sbuf, isem)
            pltpu.async_copy(dst_hbm.at[pl.ds(off, CHUNK)], dbuf, isem)

        def idx_wait(sbuf, dbuf, isem):
            pltpu.make_async_copy(src_hbm.at[pl.ds(0, CHUNK)], sbuf, isem).wait()
            pltpu.make_async_copy(dst_hbm.at[pl.ds(0, CHUNK)], dbuf, isem).wait()

        idx_load(0, src_a, dst_a, isem_a)
        idx_wait(src_a, dst_a, isem_a)
        pltpu.async_copy(vals_hbm.at[src_a], rows_a, sem_a)
        idx_load(1, src_b, dst_b, isem_b)
        plsc.subcore_barrier()

        def body(i, carry):
            j = 2 * i
            idx_wait(src_b, dst_b, isem_b)
            pltpu.async_copy(vals_hbm.at[src_b], rows_b, sem_b)
            pltpu.make_async_copy(vals_hbm.at[src_a], rows_a, sem_a).wait()
            pltpu.sync_copy(rows_a, acc.at[dst_a], add=True)

            @pl.when(i < NCH // 2 - 1)
            def _():
                idx_load(j + 2, src_a, dst_a, isem_a)
                idx_wait(src_a, dst_a, isem_a)
                pltpu.async_copy(vals_hbm.at[src_a], rows_a, sem_a)

            pltpu.make_async_copy(vals_hbm.at[src_b], rows_b, sem_b).wait()
            pltpu.sync_copy(rows_b, acc.at[dst_b], add=True)

            @pl.when(i < NCH // 2 - 1)
            def _():
                idx_load(j + 3, src_b, dst_b, isem_b)

            return carry

        lax.fori_loop(0, NCH // 2, body, 0)
        plsc.subcore_barrier()
        pltpu.sync_copy(acc.at[pl.ds(sid * RP, RP)],
                        out_hbm.at[pl.ds(cid * NP + sid * RP, RP)])

    return seg_sum


_seg_sum_full = _make_seg_sum(H)


def _dense_body(x_ref, p0_ref, p1_ref, wa_ref, wb_ref, b_ref, o_ref):
    z = jnp.dot(x_ref[...], wa_ref[...], preferred_element_type=jnp.float32)
    z = z + jnp.dot(p0_ref[...] + p1_ref[...], wb_ref[...],
                    preferred_element_type=jnp.float32)
    z = z + b_ref[...]
    o_ref[...] = jnp.maximum(z, 0.0)


def _dense_relu(x, p0, p1, wa, wb, b):
    grid = (N // BLK,)
    return pl.pallas_call(
        _dense_body,
        grid=grid,
        in_specs=[
            pl.BlockSpec((BLK, F), lambda i: (i, 0)),
            pl.BlockSpec((BLK, F), lambda i: (i, 0)),
            pl.BlockSpec((BLK, F), lambda i: (i, 0)),
            pl.BlockSpec((F, H), lambda i: (0, 0)),
            pl.BlockSpec((F, H), lambda i: (0, 0)),
            pl.BlockSpec((1, H), lambda i: (0, 0)),
        ],
        out_specs=pl.BlockSpec((BLK, H), lambda i: (i, 0)),
        out_shape=jax.ShapeDtypeStruct((N, H), jnp.float32),
    )(x, p0, p1, wa, wb, b.reshape(1, H))


def _final_body(x_ref, q0_ref, q1_ref, wa_ref, wb_ref, b_ref, o_ref):
    z = jnp.dot(x_ref[...], wa_ref[...], preferred_element_type=jnp.float32)
    z = z + jnp.dot(q0_ref[...] + q1_ref[...], wb_ref[...],
                    preferred_element_type=jnp.float32)
    z = z + b_ref[...]
    col = jax.lax.broadcasted_iota(jnp.int32, (BLK, CP), 1)
    z = jnp.where(col < C, z, -jnp.inf)
    m = jnp.max(z, axis=-1, keepdims=True)
    lse = jnp.log(jnp.sum(jnp.exp(z - m), axis=-1, keepdims=True))
    o_ref[...] = z - m - lse


def _final(h, q0, q1, wa, wb, b):
    grid = (N // BLK,)
    return pl.pallas_call(
        _final_body,
        grid=grid,
        in_specs=[
            pl.BlockSpec((BLK, H), lambda i: (i, 0)),
            pl.BlockSpec((BLK, H), lambda i: (i, 0)),
            pl.BlockSpec((BLK, H), lambda i: (i, 0)),
            pl.BlockSpec((H, CP), lambda i: (0, 0)),
            pl.BlockSpec((H, CP), lambda i: (0, 0)),
            pl.BlockSpec((1, CP), lambda i: (0, 0)),
        ],
        out_specs=pl.BlockSpec((BLK, CP), lambda i: (i, 0)),
        out_shape=jax.ShapeDtypeStruct((N, CP), jnp.float32),
    )(h, q0, q1, wa, wb, b)


def kernel(x, edge_index, W1, b1, W2, b2, W3, b3):
    ar = jnp.arange(E2 - E, dtype=jnp.int32)
    src = jnp.concatenate([edge_index[0], ar % N])
    dst = jnp.concatenate([edge_index[1], N + ar % (NP - N)])

    W1a, W1b = W1[:F], W1[F:]
    W2a, W2b = W2[:H], W2[H:]
    W3a = jnp.pad(W3[:H], ((0, 0), (0, CP - C)))
    W3b = jnp.pad(W3[H:], ((0, 0), (0, CP - C)))
    b3p = jnp.pad(b3, (0, CP - C)).reshape(1, CP)

    zeros_h = jnp.zeros((NP, H), jnp.float32)

    p1 = _seg_sum_full(x, src, dst, zeros_h)
    h1 = _dense_relu(x, p1[:N], p1[NP:NP + N], W1a, W1b, b1)

    p2 = _seg_sum_full(h1, src, dst, zeros_h)
    h2 = _dense_relu(h1, p2[:N], p2[NP:NP + N], W2a, W2b, b2)

    p3 = _seg_sum_full(h2, src, dst, zeros_h)
    out = _final(h2, p3[:N], p3[NP:NP + N], W3a, W3b, b3p)
    return out[:, :C]


# speedup vs baseline: 10.9024x; 1.1161x over previous
import functools
import jax
import jax.numpy as jnp
from jax import lax
from jax.experimental import pallas as pl
from jax.experimental.pallas import tpu as pltpu
from jax.experimental.pallas import tpu_sc as plsc

N = 10000
E = 320000
F = 128
H = 128
C = 47
CP = 64

BLK = 1000

_info = plsc.get_sparse_core_info()
NC = _info.num_cores
NS = _info.num_subcores
NW = NC * NS
CHUNK = 128
NCH = 80
E2 = NW * NCH * CHUNK
NP = 10240
RP = NP // NS


def _make_seg_sum(D):
    mesh = plsc.VectorSubcoreMesh(core_axis_name="c", subcore_axis_name="s")

    @functools.partial(
        pl.kernel,
        mesh=mesh,
        out_type=jax.ShapeDtypeStruct((NC * NP, D), jnp.float32),
        scratch_types=[
            pltpu.VMEM((NCH, CHUNK), jnp.int32),
            pltpu.VMEM((CHUNK,), jnp.int32),
            pltpu.VMEM((CHUNK,), jnp.int32),
            pltpu.VMEM((CHUNK,), jnp.int32),
            pltpu.VMEM((CHUNK,), jnp.int32),
            pltpu.VMEM((CHUNK, D), jnp.float32),
            pltpu.VMEM((CHUNK, D), jnp.float32),
            pltpu.VMEM_SHARED((NP, D), jnp.float32),
            pltpu.SemaphoreType.DMA,
            pltpu.SemaphoreType.DMA,
            pltpu.SemaphoreType.DMA,
            pltpu.SemaphoreType.DMA,
            pltpu.SemaphoreType.DMA,
            pltpu.SemaphoreType.DMA,
            pltpu.SemaphoreType.DMA,
            pltpu.SemaphoreType.DMA,
        ],
    )
    def seg_sum(vals_hbm, src_hbm, dst_hbm, zeros_hbm, out_hbm,
                dst_slab, s0, s1, s2, s3, r0, r1, acc,
                g0, g1, c0, c1, i0, i1, i2, i3):
        srcb = [s0, s1, s2, s3]
        rows = [r0, r1]
        gsem = [g0, g1]
        ssem = [c0, c1]
        isem = [i0, i1, i2, i3]
        cid = lax.axis_index("c")
        sid = lax.axis_index("s")
        wid = sid * NC + cid
        base = wid * (NCH * CHUNK)

        def src_load(j, sl):
            pltpu.async_copy(src_hbm.at[pl.ds(base + j * CHUNK, CHUNK)],
                             srcb[sl], isem[sl])

        def src_wait(sl):
            pltpu.make_async_copy(src_hbm.at[pl.ds(0, CHUNK)],
                                  srcb[sl], isem[sl]).wait()

        def gather(sl, b):
            pltpu.async_copy(vals_hbm.at[srcb[sl]], rows[b], gsem[b])

        def gather_wait(sl, b):
            pltpu.make_async_copy(vals_hbm.at[srcb[sl]],
                                  rows[b], gsem[b]).wait()

        def scatter(j, b):
            pltpu.async_copy(rows[b], acc.at[dst_slab.at[j]],
                             ssem[b], add=True)

        def scatter_wait(b):
            pltpu.make_async_copy(rows[b], acc.at[dst_slab.at[0]],
                                  ssem[b]).wait()

        pltpu.sync_copy(zeros_hbm.at[pl.ds(sid * RP, RP)],
                        acc.at[pl.ds(sid * RP, RP)])
        pltpu.sync_copy(dst_hbm.at[wid], dst_slab)
        src_load(0, 0)
        src_load(1, 1)
        src_load(2, 2)
        src_wait(0)
        gather(0, 0)
        src_load(3, 3)
        plsc.subcore_barrier()

        def body(i, carry):
            for u in range(4):
                j = 4 * i + u + 1

                def gather_side(with_ssem_wait, u=u):
                    if with_ssem_wait:
                        scatter_wait((u + 1) % 2)
                    src_wait((u + 1) % 4)
                    gather((u + 1) % 4, (u + 1) % 2)

                if u == 3:
                    @pl.when(i < NCH // 4 - 1)
                    def _(gs=gather_side):
                        gs(True)
                elif u == 0:
                    @pl.when(i > 0)
                    def _(gs=gather_side):
                        gs(True)

                    @pl.when(i == 0)
                    def _(gs=gather_side):
                        gs(False)
                else:
                    gather_side(True)

                gather_wait(u % 4, u % 2)
                scatter(j - 1, u % 2)

                @pl.when(j + 3 < NCH)
                def _(j=j, u=u):
                    src_load(j + 3, u % 4)
            return carry

        lax.fori_loop(0, NCH // 4, body, 0)
        scatter_wait(0)
        scatter_wait(1)
        plsc.subcore_barrier()
        pltpu.sync_copy(acc.at[pl.ds(sid * RP, RP)],
                        out_hbm.at[pl.ds(cid * NP + sid * RP, RP)])

    return seg_sum


_seg_sum_full = _make_seg_sum(H)


def _dense_body(x_ref, p0_ref, p1_ref, wa_ref, wb_ref, b_ref, o_ref):
    z = jnp.dot(x_ref[...], wa_ref[...], preferred_element_type=jnp.float32)
    z = z + jnp.dot(p0_ref[...] + p1_ref[...], wb_ref[...],
                    preferred_element_type=jnp.float32)
    z = z + b_ref[...]
    o_ref[...] = jnp.maximum(z, 0.0)


def _dense_relu(x, p0, p1, wa, wb, b):
    grid = (N // BLK,)
    return pl.pallas_call(
        _dense_body,
        grid=grid,
        in_specs=[
            pl.BlockSpec((BLK, F), lambda i: (i, 0)),
            pl.BlockSpec((BLK, F), lambda i: (i, 0)),
            pl.BlockSpec((BLK, F), lambda i: (i, 0)),
            pl.BlockSpec((F, H), lambda i: (0, 0)),
            pl.BlockSpec((F, H), lambda i: (0, 0)),
            pl.BlockSpec((1, H), lambda i: (0, 0)),
        ],
        out_specs=pl.BlockSpec((BLK, H), lambda i: (i, 0)),
        out_shape=jax.ShapeDtypeStruct((N, H), jnp.float32),
    )(x, p0, p1, wa, wb, b.reshape(1, H))


def _final_body(x_ref, q0_ref, q1_ref, wa_ref, wb_ref, b_ref, o_ref):
    z = jnp.dot(x_ref[...], wa_ref[...], preferred_element_type=jnp.float32)
    z = z + jnp.dot(q0_ref[...] + q1_ref[...], wb_ref[...],
                    preferred_element_type=jnp.float32)
    z = z + b_ref[...]
    col = jax.lax.broadcasted_iota(jnp.int32, (BLK, CP), 1)
    z = jnp.where(col < C, z, -jnp.inf)
    m = jnp.max(z, axis=-1, keepdims=True)
    lse = jnp.log(jnp.sum(jnp.exp(z - m), axis=-1, keepdims=True))
    o_ref[...] = z - m - lse


def _final(h, q0, q1, wa, wb, b):
    grid = (N // BLK,)
    return pl.pallas_call(
        _final_body,
        grid=grid,
        in_specs=[
            pl.BlockSpec((BLK, H), lambda i: (i, 0)),
            pl.BlockSpec((BLK, H), lambda i: (i, 0)),
            pl.BlockSpec((BLK, H), lambda i: (i, 0)),
            pl.BlockSpec((H, CP), lambda i: (0, 0)),
            pl.BlockSpec((H, CP), lambda i: (0, 0)),
            pl.BlockSpec((1, CP), lambda i: (0, 0)),
        ],
        out_specs=pl.BlockSpec((BLK, CP), lambda i: (i, 0)),
        out_shape=jax.ShapeDtypeStruct((N, CP), jnp.float32),
    )(h, q0, q1, wa, wb, b)


def kernel(x, edge_index, W1, b1, W2, b2, W3, b3):
    ar = jnp.arange(E2 - E, dtype=jnp.int32)
    src = jnp.concatenate([edge_index[0], ar % N])
    dst = jnp.concatenate([edge_index[1], N + ar % (NP - N)]).reshape(NW, NCH, CHUNK)

    W1a, W1b = W1[:F], W1[F:]
    W2a, W2b = W2[:H], W2[H:]
    W3a = jnp.pad(W3[:H], ((0, 0), (0, CP - C)))
    W3b = jnp.pad(W3[H:], ((0, 0), (0, CP - C)))
    b3p = jnp.pad(b3, (0, CP - C)).reshape(1, CP)

    zeros_h = jnp.zeros((NP, H), jnp.float32)

    p1 = _seg_sum_full(x, src, dst, zeros_h)
    h1 = _dense_relu(x, p1[:N], p1[NP:NP + N], W1a, W1b, b1)

    p2 = _seg_sum_full(h1, src, dst, zeros_h)
    h2 = _dense_relu(h1, p2[:N], p2[NP:NP + N], W2a, W2b, b2)

    p3 = _seg_sum_full(h2, src, dst, zeros_h)
    out = _final(h2, p3[:N], p3[NP:NP + N], W3a, W3b, b3p)
    return out[:, :C]


# speedup vs baseline: 11.0343x; 1.0121x over previous
import functools
import jax
import jax.numpy as jnp
from jax import lax
from jax.experimental import pallas as pl
from jax.experimental.pallas import tpu as pltpu
from jax.experimental.pallas import tpu_sc as plsc

N = 10000
E = 320000
F = 128
H = 128
C = 47
CP = 64

BLK = 1000

_info = plsc.get_sparse_core_info()
NC = _info.num_cores
NS = _info.num_subcores
NW = NC * NS
CHUNK = 128
NCH = 80
E2 = NW * NCH * CHUNK
NP = 10240
RP = NP // NS


def _make_seg_sum(D):
    mesh = plsc.VectorSubcoreMesh(core_axis_name="c", subcore_axis_name="s")

    @functools.partial(
        pl.kernel,
        mesh=mesh,
        out_type=jax.ShapeDtypeStruct((NC * NP, D), jnp.float32),
        scratch_types=[
            pltpu.VMEM((NCH, CHUNK), jnp.int32),
            pltpu.VMEM((CHUNK,), jnp.int32),
            pltpu.VMEM((CHUNK,), jnp.int32),
            pltpu.VMEM((CHUNK,), jnp.int32),
            pltpu.VMEM((CHUNK,), jnp.int32),
            pltpu.VMEM((CHUNK, D), jnp.float32),
            pltpu.VMEM((CHUNK, D), jnp.float32),
            pltpu.VMEM_SHARED((NP, D), jnp.float32),
            pltpu.SemaphoreType.DMA,
            pltpu.SemaphoreType.DMA,
            pltpu.SemaphoreType.DMA,
            pltpu.SemaphoreType.DMA,
            pltpu.SemaphoreType.DMA,
            pltpu.SemaphoreType.DMA,
            pltpu.SemaphoreType.DMA,
            pltpu.SemaphoreType.DMA,
        ],
    )
    def seg_sum(vals_hbm, src_hbm, dst_hbm, zeros_hbm, out_hbm,
                dst_slab, s0, s1, s2, s3, r0, r1, acc,
                g0, g1, c0, c1, i0, i1, i2, i3):
        srcb = [s0, s1, s2, s3]
        rows = [r0, r1]
        gsem = [g0, g1]
        ssem = [c0, c1]
        isem = [i0, i1, i2, i3]
        cid = lax.axis_index("c")
        sid = lax.axis_index("s")
        wid = sid * NC + cid
        base = wid * (NCH * CHUNK)

        def src_load(j, sl):
            pltpu.async_copy(src_hbm.at[pl.ds(base + j * CHUNK, CHUNK)],
                             srcb[sl], isem[sl])

        def src_wait(sl):
            pltpu.make_async_copy(src_hbm.at[pl.ds(0, CHUNK)],
                                  srcb[sl], isem[sl]).wait()

        def gather(sl, b):
            pltpu.async_copy(vals_hbm.at[srcb[sl]], rows[b], gsem[b])

        def gather_wait(sl, b):
            pltpu.make_async_copy(vals_hbm.at[srcb[sl]],
                                  rows[b], gsem[b]).wait()

        def scatter(j, b):
            pltpu.async_copy(rows[b], acc.at[dst_slab.at[j]],
                             ssem[b], add=True)

        def scatter_wait(b):
            pltpu.make_async_copy(rows[b], acc.at[dst_slab.at[0]],
                                  ssem[b]).wait()

        zcp0 = pltpu.make_async_copy(
            zeros_hbm.at[pl.ds(sid * RP, RP // 2)],
            acc.at[pl.ds(sid * RP, RP // 2)], c0)
        zcp0.start()
        zcp1 = pltpu.make_async_copy(
            zeros_hbm.at[pl.ds(sid * RP + RP // 2, RP // 2)],
            acc.at[pl.ds(sid * RP + RP // 2, RP // 2)], c1)
        zcp1.start()
        src_load(0, 0)
        src_load(1, 1)
        src_load(2, 2)
        pltpu.sync_copy(dst_hbm.at[wid], dst_slab)
        src_wait(0)
        gather(0, 0)
        src_load(3, 3)
        zcp0.wait()
        zcp1.wait()
        plsc.subcore_barrier()

        def body(i, carry):
            for u in range(4):
                j = 4 * i + u + 1

                def gather_side(with_ssem_wait, u=u):
                    if with_ssem_wait:
                        scatter_wait((u + 1) % 2)
                    src_wait((u + 1) % 4)
                    gather((u + 1) % 4, (u + 1) % 2)

                if u == 3:
                    @pl.when(i < NCH // 4 - 1)
                    def _(gs=gather_side):
                        gs(True)
                elif u == 0:
                    @pl.when(i > 0)
                    def _(gs=gather_side):
                        gs(True)

                    @pl.when(i == 0)
                    def _(gs=gather_side):
                        gs(False)
                else:
                    gather_side(True)

                gather_wait(u % 4, u % 2)
                scatter(j - 1, u % 2)

                @pl.when(j + 3 < NCH)
                def _(j=j, u=u):
                    src_load(j + 3, u % 4)
            return carry

        lax.fori_loop(0, NCH // 4, body, 0)
        scatter_wait(0)
        scatter_wait(1)
        plsc.subcore_barrier()
        pltpu.sync_copy(acc.at[pl.ds(sid * RP, RP)],
                        out_hbm.at[pl.ds(cid * NP + sid * RP, RP)])

    return seg_sum


_seg_sum_full = _make_seg_sum(H)


def _dense_body(x_ref, p0_ref, p1_ref, wa_ref, wb_ref, b_ref, o_ref):
    z = jnp.dot(x_ref[...], wa_ref[...], preferred_element_type=jnp.float32)
    z = z + jnp.dot(p0_ref[...] + p1_ref[...], wb_ref[...],
                    preferred_element_type=jnp.float32)
    z = z + b_ref[...]
    o_ref[...] = jnp.maximum(z, 0.0)


def _dense_relu(x, p0, p1, wa, wb, b):
    grid = (N // BLK,)
    return pl.pallas_call(
        _dense_body,
        grid=grid,
        in_specs=[
            pl.BlockSpec((BLK, F), lambda i: (i, 0)),
            pl.BlockSpec((BLK, F), lambda i: (i, 0)),
            pl.BlockSpec((BLK, F), lambda i: (i, 0)),
            pl.BlockSpec((F, H), lambda i: (0, 0)),
            pl.BlockSpec((F, H), lambda i: (0, 0)),
            pl.BlockSpec((1, H), lambda i: (0, 0)),
        ],
        out_specs=pl.BlockSpec((BLK, H), lambda i: (i, 0)),
        out_shape=jax.ShapeDtypeStruct((N, H), jnp.float32),
    )(x, p0, p1, wa, wb, b.reshape(1, H))


def _final_body(x_ref, q0_ref, q1_ref, wa_ref, wb_ref, b_ref, o_ref):
    z = jnp.dot(x_ref[...], wa_ref[...], preferred_element_type=jnp.float32)
    z = z + jnp.dot(q0_ref[...] + q1_ref[...], wb_ref[...],
                    preferred_element_type=jnp.float32)
    z = z + b_ref[...]
    col = jax.lax.broadcasted_iota(jnp.int32, (BLK, CP), 1)
    z = jnp.where(col < C, z, -jnp.inf)
    m = jnp.max(z, axis=-1, keepdims=True)
    lse = jnp.log(jnp.sum(jnp.exp(z - m), axis=-1, keepdims=True))
    o_ref[...] = z - m - lse


def _final(h, q0, q1, wa, wb, b):
    grid = (N // BLK,)
    return pl.pallas_call(
        _final_body,
        grid=grid,
        in_specs=[
            pl.BlockSpec((BLK, H), lambda i: (i, 0)),
            pl.BlockSpec((BLK, H), lambda i: (i, 0)),
            pl.BlockSpec((BLK, H), lambda i: (i, 0)),
            pl.BlockSpec((H, CP), lambda i: (0, 0)),
            pl.BlockSpec((H, CP), lambda i: (0, 0)),
            pl.BlockSpec((1, CP), lambda i: (0, 0)),
        ],
        out_specs=pl.BlockSpec((BLK, CP), lambda i: (i, 0)),
        out_shape=jax.ShapeDtypeStruct((N, CP), jnp.float32),
    )(h, q0, q1, wa, wb, b)


def kernel(x, edge_index, W1, b1, W2, b2, W3, b3):
    ar = jnp.arange(E2 - E, dtype=jnp.int32)
    src = jnp.concatenate([edge_index[0], ar % N])
    dst = jnp.concatenate([edge_index[1], N + ar % (NP - N)]).reshape(NW, NCH, CHUNK)

    W1a, W1b = W1[:F], W1[F:]
    W2a, W2b = W2[:H], W2[H:]
    W3a = jnp.pad(W3[:H], ((0, 0), (0, CP - C)))
    W3b = jnp.pad(W3[H:], ((0, 0), (0, CP - C)))
    b3p = jnp.pad(b3, (0, CP - C)).reshape(1, CP)

    zeros_h = jnp.zeros((NP, H), jnp.float32)

    p1 = _seg_sum_full(x, src, dst, zeros_h)
    h1 = _dense_relu(x, p1[:N], p1[NP:NP + N], W1a, W1b, b1)

    p2 = _seg_sum_full(h1, src, dst, zeros_h)
    h2 = _dense_relu(h1, p2[:N], p2[NP:NP + N], W2a, W2b, b2)

    p3 = _seg_sum_full(h2, src, dst, zeros_h)
    out = _final(h2, p3[:N], p3[NP:NP + N], W3a, W3b, b3p)
    return out[:, :C]


# speedup vs baseline: 11.3579x; 1.0293x over previous
import functools
import jax
import jax.numpy as jnp
from jax import lax
from jax.experimental import pallas as pl
from jax.experimental.pallas import tpu as pltpu
from jax.experimental.pallas import tpu_sc as plsc

N = 10000
E = 320000
F = 128
H = 128
C = 47
CP = 64

BLK = 5000

_info = plsc.get_sparse_core_info()
NC = _info.num_cores
NS = _info.num_subcores
NW = NC * NS
CHUNK = 128
NCH = 80
E2 = NW * NCH * CHUNK
NP = 10240
RP = NP // NS


def _make_seg_sum(D):
    mesh = plsc.VectorSubcoreMesh(core_axis_name="c", subcore_axis_name="s")

    @functools.partial(
        pl.kernel,
        mesh=mesh,
        out_type=jax.ShapeDtypeStruct((NC * NP, D), jnp.float32),
        scratch_types=[
            pltpu.VMEM((NCH, CHUNK), jnp.int32),
            pltpu.VMEM((CHUNK,), jnp.int32),
            pltpu.VMEM((CHUNK,), jnp.int32),
            pltpu.VMEM((CHUNK,), jnp.int32),
            pltpu.VMEM((CHUNK,), jnp.int32),
            pltpu.VMEM((CHUNK, D), jnp.float32),
            pltpu.VMEM((CHUNK, D), jnp.float32),
            pltpu.VMEM_SHARED((NP, D), jnp.float32),
            pltpu.SemaphoreType.DMA,
            pltpu.SemaphoreType.DMA,
            pltpu.SemaphoreType.DMA,
            pltpu.SemaphoreType.DMA,
            pltpu.SemaphoreType.DMA,
            pltpu.SemaphoreType.DMA,
            pltpu.SemaphoreType.DMA,
            pltpu.SemaphoreType.DMA,
        ],
    )
    def seg_sum(vals_hbm, src_hbm, dst_hbm, zeros_hbm, out_hbm,
                dst_slab, s0, s1, s2, s3, r0, r1, acc,
                g0, g1, c0, c1, i0, i1, i2, i3):
        srcb = [s0, s1, s2, s3]
        rows = [r0, r1]
        gsem = [g0, g1]
        ssem = [c0, c1]
        isem = [i0, i1, i2, i3]
        cid = lax.axis_index("c")
        sid = lax.axis_index("s")
        wid = sid * NC + cid
        base = wid * (NCH * CHUNK)

        def src_load(j, sl):
            pltpu.async_copy(src_hbm.at[pl.ds(base + j * CHUNK, CHUNK)],
                             srcb[sl], isem[sl])

        def src_wait(sl):
            pltpu.make_async_copy(src_hbm.at[pl.ds(0, CHUNK)],
                                  srcb[sl], isem[sl]).wait()

        def gather(sl, b):
            pltpu.async_copy(vals_hbm.at[srcb[sl]], rows[b], gsem[b])

        def gather_wait(sl, b):
            pltpu.make_async_copy(vals_hbm.at[srcb[sl]],
                                  rows[b], gsem[b]).wait()

        def scatter(j, b):
            pltpu.async_copy(rows[b], acc.at[dst_slab.at[j]],
                             ssem[b], add=True)

        def scatter_wait(b):
            pltpu.make_async_copy(rows[b], acc.at[dst_slab.at[0]],
                                  ssem[b]).wait()

        zcp0 = pltpu.make_async_copy(
            zeros_hbm.at[pl.ds(sid * RP, RP // 2)],
            acc.at[pl.ds(sid * RP, RP // 2)], c0)
        zcp0.start()
        zcp1 = pltpu.make_async_copy(
            zeros_hbm.at[pl.ds(sid * RP + RP // 2, RP // 2)],
            acc.at[pl.ds(sid * RP + RP // 2, RP // 2)], c1)
        zcp1.start()
        src_load(0, 0)
        src_load(1, 1)
        src_load(2, 2)
        pltpu.sync_copy(dst_hbm.at[wid], dst_slab)
        src_wait(0)
        gather(0, 0)
        src_load(3, 3)
        zcp0.wait()
        zcp1.wait()
        plsc.subcore_barrier()

        def body(i, carry):
            for u in range(4):
                j = 4 * i + u + 1

                def gather_side(with_ssem_wait, u=u):
                    if with_ssem_wait:
                        scatter_wait((u + 1) % 2)
                    src_wait((u + 1) % 4)
                    gather((u + 1) % 4, (u + 1) % 2)

                if u == 3:
                    @pl.when(i < NCH // 4 - 1)
                    def _(gs=gather_side):
                        gs(True)
                elif u == 0:
                    @pl.when(i > 0)
                    def _(gs=gather_side):
                        gs(True)

                    @pl.when(i == 0)
                    def _(gs=gather_side):
                        gs(False)
                else:
                    gather_side(True)

                gather_wait(u % 4, u % 2)
                scatter(j - 1, u % 2)

                @pl.when(j + 3 < NCH)
                def _(j=j, u=u):
                    src_load(j + 3, u % 4)
            return carry

        lax.fori_loop(0, NCH // 4, body, 0)
        scatter_wait(0)
        scatter_wait(1)
        plsc.subcore_barrier()
        pltpu.sync_copy(acc.at[pl.ds(sid * RP, RP)],
                        out_hbm.at[pl.ds(cid * NP + sid * RP, RP)])

    return seg_sum


_seg_sum_full = _make_seg_sum(H)


def _dense_body(x_ref, p0_ref, p1_ref, wa_ref, wb_ref, b_ref, o_ref):
    z = jnp.dot(x_ref[...], wa_ref[...], preferred_element_type=jnp.float32)
    z = z + jnp.dot(p0_ref[...] + p1_ref[...], wb_ref[...],
                    preferred_element_type=jnp.float32)
    z = z + b_ref[...]
    o_ref[...] = jnp.maximum(z, 0.0)


def _dense_relu(x, p0, p1, wa, wb, b):
    grid = (N // BLK,)
    return pl.pallas_call(
        _dense_body,
        grid=grid,
        in_specs=[
            pl.BlockSpec((BLK, F), lambda i: (i, 0)),
            pl.BlockSpec((BLK, F), lambda i: (i, 0)),
            pl.BlockSpec((BLK, F), lambda i: (i, 0)),
            pl.BlockSpec((F, H), lambda i: (0, 0)),
            pl.BlockSpec((F, H), lambda i: (0, 0)),
            pl.BlockSpec((1, H), lambda i: (0, 0)),
        ],
        out_specs=pl.BlockSpec((BLK, H), lambda i: (i, 0)),
        out_shape=jax.ShapeDtypeStruct((N, H), jnp.float32),
    )(x, p0, p1, wa, wb, b.reshape(1, H))


def _final_body(x_ref, q0_ref, q1_ref, wa_ref, wb_ref, b_ref, o_ref):
    z = jnp.dot(x_ref[...], wa_ref[...], preferred_element_type=jnp.float32)
    z = z + jnp.dot(q0_ref[...] + q1_ref[...], wb_ref[...],
                    preferred_element_type=jnp.float32)
    z = z + b_ref[...]
    col = jax.lax.broadcasted_iota(jnp.int32, (BLK, CP), 1)
    z = jnp.where(col < C, z, -jnp.inf)
    m = jnp.max(z, axis=-1, keepdims=True)
    lse = jnp.log(jnp.sum(jnp.exp(z - m), axis=-1, keepdims=True))
    o_ref[...] = z - m - lse


def _final(h, q0, q1, wa, wb, b):
    grid = (N // BLK,)
    return pl.pallas_call(
        _final_body,
        grid=grid,
        in_specs=[
            pl.BlockSpec((BLK, H), lambda i: (i, 0)),
            pl.BlockSpec((BLK, H), lambda i: (i, 0)),
            pl.BlockSpec((BLK, H), lambda i: (i, 0)),
            pl.BlockSpec((H, CP), lambda i: (0, 0)),
            pl.BlockSpec((H, CP), lambda i: (0, 0)),
            pl.BlockSpec((1, CP), lambda i: (0, 0)),
        ],
        out_specs=pl.BlockSpec((BLK, CP), lambda i: (i, 0)),
        out_shape=jax.ShapeDtypeStruct((N, CP), jnp.float32),
    )(h, q0, q1, wa, wb, b)


def kernel(x, edge_index, W1, b1, W2, b2, W3, b3):
    ar = jnp.arange(E2 - E, dtype=jnp.int32)
    src = jnp.concatenate([edge_index[0], ar % N])
    dst = jnp.concatenate([edge_index[1], N + ar % (NP - N)]).reshape(NW, NCH, CHUNK)

    W1a, W1b = W1[:F], W1[F:]
    W2a, W2b = W2[:H], W2[H:]
    W3a = jnp.pad(W3[:H], ((0, 0), (0, CP - C)))
    W3b = jnp.pad(W3[H:], ((0, 0), (0, CP - C)))
    b3p = jnp.pad(b3, (0, CP - C)).reshape(1, CP)

    zeros_h = jnp.zeros((NP, H), jnp.float32)

    p1 = _seg_sum_full(x, src, dst, zeros_h)
    h1 = _dense_relu(x, p1[:N], p1[NP:NP + N], W1a, W1b, b1)

    p2 = _seg_sum_full(h1, src, dst, zeros_h)
    h2 = _dense_relu(h1, p2[:N], p2[NP:NP + N], W2a, W2b, b2)

    p3 = _seg_sum_full(h2, src, dst, zeros_h)
    out = _final(h2, p3[:N], p3[NP:NP + N], W3a, W3b, b3p)
    return out[:, :C]
